# baseline (speedup 1.0000x reference)
import functools

import jax
import jax.numpy as jnp
from jax import lax
from jax.experimental import pallas as pl
from jax.experimental.pallas import tpu as pltpu
from jax.experimental.pallas import tpu_sc as plsc

N_LABELS = 1000
BATCH = 16384
MASKED = 200
SC_COLS = 128
NC, NS, L = 2, 16, 16
NW = NC * NS
ROWS_PER_W = BATCH // NW
CHUNK = 128
N_CHUNKS = ROWS_PER_W // CHUNK
NBUF = 4
PDIST = 2
SC_OFFS = tuple(range(0, SC_COLS, L))

TC_COLB = 128
TC_NCOLB = 7


def _make_sc_call():
    mesh = plsc.VectorSubcoreMesh(core_axis_name="c", subcore_axis_name="s")

    @functools.partial(
        pl.kernel,
        mesh=mesh,
        out_type=jax.ShapeDtypeStruct((BATCH, SC_COLS), jnp.float32),
        scratch_types=[
            pltpu.VMEM((NBUF, CHUNK, SC_COLS), jnp.float32),
            pltpu.SemaphoreType.DMA((NBUF,)),
            pltpu.SemaphoreType.DMA((NBUF,)),
        ],
        compiler_params=pltpu.CompilerParams(
            use_tc_tiling_on_sc=False, needs_layout_passes=False),
    )
    def run(x_hbm, out_hbm, bufs, sin, sout):
        wid = lax.axis_index("s") * NC + lax.axis_index("c")
        base_row = wid * ROWS_PER_W
        lane = lax.broadcasted_iota(jnp.int32, (L,), 0)
        group_base = lane & jnp.int32(-4)

        def row_slice(ci):
            return pl.ds(base_row + ci * CHUNK, CHUNK)

        def start_in(ci, b):
            pltpu.async_copy(
                x_hbm.at[row_slice(ci), pl.ds(0, SC_COLS)], bufs.at[b],
                sin.at[b])

        def wait_in(ci, b):
            pltpu.make_async_copy(
                x_hbm.at[row_slice(ci), pl.ds(0, SC_COLS)], bufs.at[b],
                sin.at[b]).wait()

        def start_out(ci, b):
            pltpu.async_copy(bufs.at[b], out_hbm.at[row_slice(ci)], sout.at[b])

        def wait_out(ci, b):
            pltpu.make_async_copy(
                bufs.at[b], out_hbm.at[row_slice(ci)], sout.at[b]).wait()

        def compute(b):
            b_vec = jnp.full((L,), b, dtype=jnp.int32)

            def row_body(r, carry):
                r_vec = jnp.full((L,), r, dtype=jnp.int32)
                for c in SC_OFFS:
                    v = bufs[b, r, pl.ds(c, L)]
                    cb = group_base + jnp.int32(c)
                    g0 = plsc.load_gather(bufs, [b_vec, r_vec, cb])
                    g1 = plsc.load_gather(bufs, [b_vec, r_vec, cb + 1])
                    g2 = plsc.load_gather(bufs, [b_vec, r_vec, cb + 2])
                    g3 = plsc.load_gather(bufs, [b_vec, r_vec, cb + 3])
                    gmax = jnp.maximum(
                        jnp.maximum(g0, g1), jnp.maximum(g2, g3))
                    bufs[b, r, pl.ds(c, L)] = jnp.where(
                        v == gmax, v, jnp.float32(-1.0))
                return carry

            lax.fori_loop(0, CHUNK, row_body, 0)

        for ci in range(PDIST):
            start_in(ci, ci % NBUF)

        def outer(g, carry):
            for b in range(NBUF):
                ci = g * NBUF + b
                wait_in(ci, b)
                compute(b)
                start_out(ci, b)
                nci = ci + PDIST
                nb = (b + PDIST) % NBUF

                @pl.when(nci < N_CHUNKS)
                def _():
                    @pl.when(ci >= PDIST)
                    def _():
                        wait_out(ci - PDIST, nb)
                    start_in(nci, nb)
            return carry

        lax.fori_loop(0, N_CHUNKS // NBUF, outer, 0)
        for x in range(N_CHUNKS - NBUF, N_CHUNKS):
            wait_out(x, x % NBUF)

    return run


_sc_call = _make_sc_call()


def _tc_stream_body(x_ref, o_ref):
    j = pl.program_id(0)

    @pl.when(j == 0)
    def _():
        v = x_ref[...]
        lane = lax.broadcasted_iota(jnp.int32, v.shape, 1)
        left1 = pltpu.roll(v, TC_COLB - 1, 1)
        right1 = pltpu.roll(v, 1, 1)
        m1 = jnp.maximum(v, jnp.where((lane & 1) == 0, left1, right1))
        left2 = pltpu.roll(m1, TC_COLB - 2, 1)
        right2 = pltpu.roll(m1, 2, 1)
        gmax = jnp.maximum(m1, jnp.where((lane & 3) < 2, left2, right2))
        keep = jnp.logical_or(v == gmax, TC_COLB + lane >= MASKED)
        o_ref[...] = jnp.where(keep, v, jnp.float32(-1.0))

    @pl.when(j != 0)
    def _():
        o_ref[...] = x_ref[...]


def _tc_stream(x):
    return pl.pallas_call(
        _tc_stream_body,
        grid=(TC_NCOLB,),
        in_specs=[pl.BlockSpec((BATCH, TC_COLB), lambda j: (0, j + 1))],
        out_specs=pl.BlockSpec((BATCH, TC_COLB), lambda j: (0, j + 1)),
        out_shape=jax.ShapeDtypeStruct((BATCH, N_LABELS), jnp.float32),
    )(x)


def _tc_paste_body(scm_ref, alias_ref, o_ref):
    o_ref[...] = scm_ref[...]


def _tc_paste(scm, out1):
    return pl.pallas_call(
        _tc_paste_body,
        grid=(4,),
        in_specs=[
            pl.BlockSpec((BATCH // 4, SC_COLS), lambda i: (i, 0)),
            pl.BlockSpec(memory_space=pl.ANY),
        ],
        out_specs=pl.BlockSpec((BATCH // 4, SC_COLS), lambda i: (i, 0)),
        out_shape=jax.ShapeDtypeStruct((BATCH, N_LABELS), jnp.float32),
        input_output_aliases={1: 0},
    )(scm, out1)


def kernel(similarities):
    scm = _sc_call(similarities)
    out1 = _tc_stream(similarities)
    return _tc_paste(scm, out1)


# speedup vs baseline: 3.8858x; 3.8858x over previous
import functools

import jax
import jax.numpy as jnp
from jax import lax
from jax.experimental import pallas as pl
from jax.experimental.pallas import tpu as pltpu
from jax.experimental.pallas import tpu_sc as plsc

N_LABELS = 1000
BATCH = 16384
MASKED = 200
SC_ROWS = 192
NC, NS, L = 2, 16, 16
NW = NC * NS

UNIT_COLS = 1024
N_STRIPES = SC_ROWS // 8
N_CCHUNK = BATCH // UNIT_COLS
UNITS_PER_W = N_STRIPES * N_CCHUNK // NW
NBUF = 4
PDIST = 2

TC_RBLK = 200
TC_CBLK = 4096


def _make_sc_call():
    mesh = plsc.VectorSubcoreMesh(core_axis_name="c", subcore_axis_name="s")

    @functools.partial(
        pl.kernel,
        mesh=mesh,
        out_type=jax.ShapeDtypeStruct((SC_ROWS, BATCH), jnp.float32),
        scratch_types=[
            pltpu.VMEM((NBUF, 8, UNIT_COLS), jnp.float32),
            pltpu.SemaphoreType.DMA((NBUF,)),
            pltpu.SemaphoreType.DMA((NBUF,)),
        ],
        compiler_params=pltpu.CompilerParams(use_tc_tiling_on_sc=True),
    )
    def run(xt_hbm, out_hbm, bufs, sin, sout):
        wid = lax.axis_index("s") * NC + lax.axis_index("c")
        ubase = wid * UNITS_PER_W

        def unit_slices(u):
            uu = ubase + u
            s = uu // N_CCHUNK
            cc = uu % N_CCHUNK
            return pl.ds(s * 8, 8), pl.ds(cc * UNIT_COLS, UNIT_COLS)

        def start_in(u, b):
            rs, cs = unit_slices(u)
            pltpu.async_copy(xt_hbm.at[rs, cs], bufs.at[b], sin.at[b])

        def wait_in(u, b):
            rs, cs = unit_slices(u)
            pltpu.make_async_copy(xt_hbm.at[rs, cs], bufs.at[b], sin.at[b]).wait()

        def start_out(u, b):
            rs, cs = unit_slices(u)
            pltpu.async_copy(bufs.at[b], out_hbm.at[rs, cs], sout.at[b])

        def wait_out(u, b):
            rs, cs = unit_slices(u)
            pltpu.make_async_copy(bufs.at[b], out_hbm.at[rs, cs], sout.at[b]).wait()

        def compute(b):
            def col_body(c16, carry):
                c = c16 * L
                for r0 in (0, 4):
                    v0 = bufs[b, r0, pl.ds(c, L)]
                    v1 = bufs[b, r0 + 1, pl.ds(c, L)]
                    v2 = bufs[b, r0 + 2, pl.ds(c, L)]
                    v3 = bufs[b, r0 + 3, pl.ds(c, L)]
                    gmax = jnp.maximum(jnp.maximum(v0, v1),
                                       jnp.maximum(v2, v3))
                    neg1 = jnp.float32(-1.0)
                    bufs[b, r0, pl.ds(c, L)] = jnp.where(v0 == gmax, v0, neg1)
                    bufs[b, r0 + 1, pl.ds(c, L)] = jnp.where(
                        v1 == gmax, v1, neg1)
                    bufs[b, r0 + 2, pl.ds(c, L)] = jnp.where(
                        v2 == gmax, v2, neg1)
                    bufs[b, r0 + 3, pl.ds(c, L)] = jnp.where(
                        v3 == gmax, v3, neg1)
                return carry

            lax.fori_loop(0, UNIT_COLS // L, col_body, 0)

        for u in range(PDIST):
            start_in(u, u % NBUF)

        def outer(g, carry):
            for b in range(NBUF):
                u = g * NBUF + b
                wait_in(u, b)
                compute(b)
                start_out(u, b)
                nu = u + PDIST
                nb = (b + PDIST) % NBUF

                @pl.when(nu < UNITS_PER_W)
                def _():
                    @pl.when(u >= PDIST)
                    def _():
                        wait_out(u - PDIST, nb)
                    start_in(nu, nb)
            return carry

        lax.fori_loop(0, UNITS_PER_W // NBUF, outer, 0)
        for x in range(UNITS_PER_W - NBUF, UNITS_PER_W):
            wait_out(x, x % NBUF)

    return run


_sc_call = _make_sc_call()


def _tc_stream_body(x_ref, o_ref):
    o_ref[...] = x_ref[...]


def _tc_stream(xt):
    return pl.pallas_call(
        _tc_stream_body,
        grid=((N_LABELS - MASKED) // TC_RBLK, BATCH // TC_CBLK),
        in_specs=[pl.BlockSpec((TC_RBLK, TC_CBLK), lambda i, j: (i + 1, j))],
        out_specs=pl.BlockSpec((TC_RBLK, TC_CBLK), lambda i, j: (i + 1, j)),
        out_shape=jax.ShapeDtypeStruct((N_LABELS, BATCH), jnp.float32),
    )(xt)


def _tc_paste_body(scm_ref, x8_ref, alias_ref, o_ref):
    o_ref[0:SC_ROWS, :] = scm_ref[...]
    v = x8_ref[...]
    row = lax.broadcasted_iota(jnp.int32, v.shape, 0)
    up1 = pltpu.roll(v, 7, 0)
    dn1 = pltpu.roll(v, 1, 0)
    m1 = jnp.maximum(v, jnp.where((row & 1) == 0, up1, dn1))
    up2 = pltpu.roll(m1, 6, 0)
    dn2 = pltpu.roll(m1, 2, 0)
    gmax = jnp.maximum(m1, jnp.where((row & 3) < 2, up2, dn2))
    o_ref[SC_ROWS:MASKED, :] = jnp.where(v == gmax, v, jnp.float32(-1.0))


def _tc_paste(scm, xt, out1):
    return pl.pallas_call(
        _tc_paste_body,
        grid=(BATCH // TC_CBLK,),
        in_specs=[
            pl.BlockSpec((SC_ROWS, TC_CBLK), lambda j: (0, j)),
            pl.BlockSpec((8, TC_CBLK), lambda j: (SC_ROWS // 8, j)),
            pl.BlockSpec(memory_space=pl.ANY),
        ],
        out_specs=pl.BlockSpec((MASKED, TC_CBLK), lambda j: (0, j)),
        out_shape=jax.ShapeDtypeStruct((N_LABELS, BATCH), jnp.float32),
        input_output_aliases={2: 0},
    )(scm, xt, out1)


def kernel(similarities):
    xt = jnp.transpose(similarities)
    scm = _sc_call(xt)
    out1 = _tc_stream(xt)
    out_t = _tc_paste(scm, xt, out1)
    return jnp.transpose(out_t)


# speedup vs baseline: 4.0635x; 1.0457x over previous
import functools

import jax
import jax.numpy as jnp
from jax import lax
from jax.experimental import pallas as pl
from jax.experimental.pallas import tpu as pltpu
from jax.experimental.pallas import tpu_sc as plsc

N_LABELS = 1000
BATCH = 16384
MASKED = 200
SC_ROWS = 192
NC, NS, L = 2, 16, 16
NW = NC * NS

UNIT_COLS = 1024
N_STRIPES = SC_ROWS // 8
N_CCHUNK = BATCH // UNIT_COLS
UNITS_PER_W = N_STRIPES * N_CCHUNK // NW
NBUF = 4
PDIST = 2

TC_RBLK = 200
TC_CBLK = 8192


def _make_sc_call():
    mesh = plsc.VectorSubcoreMesh(core_axis_name="c", subcore_axis_name="s")

    @functools.partial(
        pl.kernel,
        mesh=mesh,
        out_type=jax.ShapeDtypeStruct((SC_ROWS, BATCH), jnp.float32),
        scratch_types=[
            pltpu.VMEM((NBUF, 8, UNIT_COLS), jnp.float32),
            pltpu.SemaphoreType.DMA((NBUF,)),
            pltpu.SemaphoreType.DMA((NBUF,)),
        ],
        compiler_params=pltpu.CompilerParams(use_tc_tiling_on_sc=True),
    )
    def run(xt_hbm, out_hbm, bufs, sin, sout):
        wid = lax.axis_index("s") * NC + lax.axis_index("c")
        ubase = wid * UNITS_PER_W

        def unit_slices(u):
            uu = ubase + u
            s = uu // N_CCHUNK
            cc = uu % N_CCHUNK
            return pl.ds(s * 8, 8), pl.ds(cc * UNIT_COLS, UNIT_COLS)

        def start_in(u, b):
            rs, cs = unit_slices(u)
            pltpu.async_copy(xt_hbm.at[rs, cs], bufs.at[b], sin.at[b])

        def wait_in(u, b):
            rs, cs = unit_slices(u)
            pltpu.make_async_copy(xt_hbm.at[rs, cs], bufs.at[b], sin.at[b]).wait()

        def start_out(u, b):
            rs, cs = unit_slices(u)
            pltpu.async_copy(bufs.at[b], out_hbm.at[rs, cs], sout.at[b])

        def wait_out(u, b):
            rs, cs = unit_slices(u)
            pltpu.make_async_copy(bufs.at[b], out_hbm.at[rs, cs], sout.at[b]).wait()

        def compute(b):
            def col_body(c16, carry):
                c = c16 * L
                for r0 in (0, 4):
                    v0 = bufs[b, r0, pl.ds(c, L)]
                    v1 = bufs[b, r0 + 1, pl.ds(c, L)]
                    v2 = bufs[b, r0 + 2, pl.ds(c, L)]
                    v3 = bufs[b, r0 + 3, pl.ds(c, L)]
                    gmax = jnp.maximum(jnp.maximum(v0, v1),
                                       jnp.maximum(v2, v3))
                    neg1 = jnp.float32(-1.0)
                    bufs[b, r0, pl.ds(c, L)] = jnp.where(v0 == gmax, v0, neg1)
                    bufs[b, r0 + 1, pl.ds(c, L)] = jnp.where(
                        v1 == gmax, v1, neg1)
                    bufs[b, r0 + 2, pl.ds(c, L)] = jnp.where(
                        v2 == gmax, v2, neg1)
                    bufs[b, r0 + 3, pl.ds(c, L)] = jnp.where(
                        v3 == gmax, v3, neg1)
                return carry

            lax.fori_loop(0, UNIT_COLS // L, col_body, 0)

        for u in range(PDIST):
            start_in(u, u % NBUF)

        def outer(g, carry):
            for b in range(NBUF):
                u = g * NBUF + b
                wait_in(u, b)
                compute(b)
                start_out(u, b)
                nu = u + PDIST
                nb = (b + PDIST) % NBUF

                @pl.when(nu < UNITS_PER_W)
                def _():
                    @pl.when(u >= PDIST)
                    def _():
                        wait_out(u - PDIST, nb)
                    start_in(nu, nb)
            return carry

        lax.fori_loop(0, UNITS_PER_W // NBUF, outer, 0)
        for x in range(UNITS_PER_W - NBUF, UNITS_PER_W):
            wait_out(x, x % NBUF)

    return run


_sc_call = _make_sc_call()


def _tc_stream_body(x_ref, o_ref):
    o_ref[...] = x_ref[...]


def _tc_stream(xt):
    return pl.pallas_call(
        _tc_stream_body,
        grid=((N_LABELS - MASKED) // TC_RBLK, BATCH // TC_CBLK),
        in_specs=[pl.BlockSpec((TC_RBLK, TC_CBLK), lambda i, j: (i + 1, j))],
        out_specs=pl.BlockSpec((TC_RBLK, TC_CBLK), lambda i, j: (i + 1, j)),
        out_shape=jax.ShapeDtypeStruct((N_LABELS, BATCH), jnp.float32),
    )(xt)


def _tc_paste_body(scm_ref, x8_ref, alias_ref, o_ref):
    o_ref[0:SC_ROWS, :] = scm_ref[...]
    v = x8_ref[...]
    row = lax.broadcasted_iota(jnp.int32, v.shape, 0)
    up1 = pltpu.roll(v, 7, 0)
    dn1 = pltpu.roll(v, 1, 0)
    m1 = jnp.maximum(v, jnp.where((row & 1) == 0, up1, dn1))
    up2 = pltpu.roll(m1, 6, 0)
    dn2 = pltpu.roll(m1, 2, 0)
    gmax = jnp.maximum(m1, jnp.where((row & 3) < 2, up2, dn2))
    o_ref[SC_ROWS:MASKED, :] = jnp.where(v == gmax, v, jnp.float32(-1.0))


def _tc_paste(scm, xt, out1):
    return pl.pallas_call(
        _tc_paste_body,
        grid=(BATCH // TC_CBLK,),
        in_specs=[
            pl.BlockSpec((SC_ROWS, TC_CBLK), lambda j: (0, j)),
            pl.BlockSpec((8, TC_CBLK), lambda j: (SC_ROWS // 8, j)),
            pl.BlockSpec(memory_space=pl.ANY),
        ],
        out_specs=pl.BlockSpec((MASKED, TC_CBLK), lambda j: (0, j)),
        out_shape=jax.ShapeDtypeStruct((N_LABELS, BATCH), jnp.float32),
        input_output_aliases={2: 0},
    )(scm, xt, out1)


def kernel(similarities):
    xt = jnp.transpose(similarities)
    scm = _sc_call(xt)
    out1 = _tc_stream(xt)
    out_t = _tc_paste(scm, xt, out1)
    return jnp.transpose(out_t)


# speedup vs baseline: 4.1667x; 1.0254x over previous
import functools

import jax
import jax.numpy as jnp
from jax import lax
from jax.experimental import pallas as pl
from jax.experimental.pallas import tpu as pltpu
from jax.experimental.pallas import tpu_sc as plsc

N_LABELS = 1000
BATCH = 16384
MASKED = 200
SC_ROWS = 192
NC, NS, L = 2, 16, 16
NW = NC * NS

UNIT_COLS = 1024
N_STRIPES = SC_ROWS // 8
N_CCHUNK = BATCH // UNIT_COLS
UNITS_PER_W = N_STRIPES * N_CCHUNK // NW
NBUF = 4
PDIST = 2

TC_RBLK = 200
TC_SBLK = 16384
TC_CBLK = 8192


def _make_sc_call():
    mesh = plsc.VectorSubcoreMesh(core_axis_name="c", subcore_axis_name="s")

    @functools.partial(
        pl.kernel,
        mesh=mesh,
        out_type=jax.ShapeDtypeStruct((SC_ROWS, BATCH), jnp.float32),
        scratch_types=[
            pltpu.VMEM((NBUF, 8, UNIT_COLS), jnp.float32),
            pltpu.SemaphoreType.DMA((NBUF,)),
            pltpu.SemaphoreType.DMA((NBUF,)),
        ],
        compiler_params=pltpu.CompilerParams(use_tc_tiling_on_sc=True),
    )
    def run(xt_hbm, out_hbm, bufs, sin, sout):
        wid = lax.axis_index("s") * NC + lax.axis_index("c")
        ubase = wid * UNITS_PER_W

        def unit_slices(u):
            uu = ubase + u
            s = uu // N_CCHUNK
            cc = uu % N_CCHUNK
            return pl.ds(s * 8, 8), pl.ds(cc * UNIT_COLS, UNIT_COLS)

        def start_in(u, b):
            rs, cs = unit_slices(u)
            pltpu.async_copy(xt_hbm.at[rs, cs], bufs.at[b], sin.at[b])

        def wait_in(u, b):
            rs, cs = unit_slices(u)
            pltpu.make_async_copy(xt_hbm.at[rs, cs], bufs.at[b], sin.at[b]).wait()

        def start_out(u, b):
            rs, cs = unit_slices(u)
            pltpu.async_copy(bufs.at[b], out_hbm.at[rs, cs], sout.at[b])

        def wait_out(u, b):
            rs, cs = unit_slices(u)
            pltpu.make_async_copy(bufs.at[b], out_hbm.at[rs, cs], sout.at[b]).wait()

        def compute(b):
            def col_body(c16, carry):
                c = c16 * L
                for r0 in (0, 4):
                    v0 = bufs[b, r0, pl.ds(c, L)]
                    v1 = bufs[b, r0 + 1, pl.ds(c, L)]
                    v2 = bufs[b, r0 + 2, pl.ds(c, L)]
                    v3 = bufs[b, r0 + 3, pl.ds(c, L)]
                    gmax = jnp.maximum(jnp.maximum(v0, v1),
                                       jnp.maximum(v2, v3))
                    neg1 = jnp.float32(-1.0)
                    bufs[b, r0, pl.ds(c, L)] = jnp.where(v0 == gmax, v0, neg1)
                    bufs[b, r0 + 1, pl.ds(c, L)] = jnp.where(
                        v1 == gmax, v1, neg1)
                    bufs[b, r0 + 2, pl.ds(c, L)] = jnp.where(
                        v2 == gmax, v2, neg1)
                    bufs[b, r0 + 3, pl.ds(c, L)] = jnp.where(
                        v3 == gmax, v3, neg1)
                return carry

            lax.fori_loop(0, UNIT_COLS // L, col_body, 0)

        for u in range(PDIST):
            start_in(u, u % NBUF)

        def outer(g, carry):
            for b in range(NBUF):
                u = g * NBUF + b
                wait_in(u, b)
                compute(b)
                start_out(u, b)
                nu = u + PDIST
                nb = (b + PDIST) % NBUF

                @pl.when(nu < UNITS_PER_W)
                def _():
                    @pl.when(u >= PDIST)
                    def _():
                        wait_out(u - PDIST, nb)
                    start_in(nu, nb)
            return carry

        lax.fori_loop(0, UNITS_PER_W // NBUF, outer, 0)
        for x in range(UNITS_PER_W - NBUF, UNITS_PER_W):
            wait_out(x, x % NBUF)

    return run


_sc_call = _make_sc_call()


def _tc_stream_body(x_ref, o_ref):
    o_ref[...] = x_ref[...]


def _tc_stream(xt):
    return pl.pallas_call(
        _tc_stream_body,
        grid=((N_LABELS - MASKED) // TC_RBLK, BATCH // TC_SBLK),
        in_specs=[pl.BlockSpec((TC_RBLK, TC_SBLK), lambda i, j: (i + 1, j))],
        out_specs=pl.BlockSpec((TC_RBLK, TC_SBLK), lambda i, j: (i + 1, j)),
        out_shape=jax.ShapeDtypeStruct((N_LABELS, BATCH), jnp.float32),
    )(xt)


def _tc_paste_body(scm_ref, x8_ref, alias_ref, o_ref):
    o_ref[0:SC_ROWS, :] = scm_ref[...]
    v = x8_ref[...]
    row = lax.broadcasted_iota(jnp.int32, v.shape, 0)
    up1 = pltpu.roll(v, 7, 0)
    dn1 = pltpu.roll(v, 1, 0)
    m1 = jnp.maximum(v, jnp.where((row & 1) == 0, up1, dn1))
    up2 = pltpu.roll(m1, 6, 0)
    dn2 = pltpu.roll(m1, 2, 0)
    gmax = jnp.maximum(m1, jnp.where((row & 3) < 2, up2, dn2))
    o_ref[SC_ROWS:MASKED, :] = jnp.where(v == gmax, v, jnp.float32(-1.0))


def _tc_paste(scm, xt, out1):
    return pl.pallas_call(
        _tc_paste_body,
        grid=(BATCH // TC_CBLK,),
        in_specs=[
            pl.BlockSpec((SC_ROWS, TC_CBLK), lambda j: (0, j)),
            pl.BlockSpec((8, TC_CBLK), lambda j: (SC_ROWS // 8, j)),
            pl.BlockSpec(memory_space=pl.ANY),
        ],
        out_specs=pl.BlockSpec((MASKED, TC_CBLK), lambda j: (0, j)),
        out_shape=jax.ShapeDtypeStruct((N_LABELS, BATCH), jnp.float32),
        input_output_aliases={2: 0},
    )(scm, xt, out1)


def kernel(similarities):
    xt = jnp.transpose(similarities)
    scm = _sc_call(xt)
    out1 = _tc_stream(xt)
    out_t = _tc_paste(scm, xt, out1)
    return jnp.transpose(out_t)
